# baseline (speedup 1.0000x reference)
import functools

import jax
import jax.numpy as jnp
from jax import lax
from jax.experimental import pallas as pl
from jax.experimental.pallas import tpu as pltpu
from jax.experimental.pallas import tpu_sc as plsc

D = 64
IB = 128
CHUNK = 512


def _make_gather(n_rows: int):
    info = plsc.get_sparse_core_info()
    nc, ns = info.num_cores, info.num_subcores
    nw = nc * ns
    per_w = n_rows // nw
    n_chunks = per_w // CHUNK
    n_sub = CHUNK // IB
    mesh = plsc.VectorSubcoreMesh(core_axis_name="c", subcore_axis_name="s")

    @functools.partial(
        pl.kernel,
        out_type=jax.ShapeDtypeStruct((n_rows, 2 * D), jnp.float32),
        mesh=mesh,
        compiler_params=pltpu.CompilerParams(
            use_tc_tiling_on_sc=False, needs_layout_passes=False),
        scratch_types=[
            pltpu.VMEM((per_w // IB, IB), jnp.int32),
            pltpu.VMEM((per_w,), jnp.float32),
            pltpu.VMEM((CHUNK, D), jnp.float32),
            pltpu.VMEM((CHUNK, D), jnp.float32),
            pltpu.SemaphoreType.DMA,
            pltpu.SemaphoreType.DMA,
            pltpu.SemaphoreType.DMA,
            pltpu.SemaphoreType.DMA,
        ],
    )
    def gather_kernel(tpad_hbm, idx_hbm, dur_hbm, out_hbm, idx_v, dur_v,
                      rows0, rows1, gsem0, gsem1, osem0, osem1):
        wid = lax.axis_index("s") * nc + lax.axis_index("c")
        base0 = pl.multiple_of(wid * per_w, CHUNK)
        rows = (rows0, rows1)
        gsem = (gsem0, gsem1)
        osem = (osem0, osem1)

        irow = pl.multiple_of(base0 // IB, 8)
        pltpu.sync_copy(idx_hbm.at[pl.ds(irow, per_w // IB)], idx_v)
        pltpu.sync_copy(dur_hbm.at[pl.ds(base0, per_w)], dur_v)

        def issue_gather(c, bi):
            for jj in range(n_sub):
                pltpu.async_copy(
                    tpad_hbm.at[idx_v.at[c * n_sub + jj]],
                    rows[bi].at[pl.ds(jj * IB, IB)],
                    gsem[bi],
                )

        def drain_gather(bi):
            pltpu.make_async_copy(
                out_hbm.at[pl.ds(0, CHUNK), 0:D], rows[bi], gsem[bi]).wait()

        def drain_out(c, bi):
            pltpu.make_async_copy(
                rows[bi], out_hbm.at[pl.ds(0, CHUNK), 0:D], osem[bi]).wait()

        def finish_chunk(c, bi):
            drain_gather(bi)
            col = jnp.full((16,), D - 1, jnp.int32)
            for k in range(CHUNK // 16):
                r = lax.iota(jnp.int32, 16) + (k * 16)
                v = dur_v[pl.ds(c * CHUNK + k * 16, 16)]
                plsc.store_scatter(rows[bi], [r, col], v)
            obase = pl.multiple_of(base0 + c * CHUNK, CHUNK)
            pltpu.async_copy(rows[bi], out_hbm.at[pl.ds(obase, CHUNK), 0:D],
                             osem[bi])

        issue_gather(0, 0)

        def pair_body(p, _):
            c0 = p * 2

            @pl.when(c0 + 1 < n_chunks)
            def _():
                pl.when(c0 >= 2)(lambda: drain_out(c0 + 1 - 2, 1))
                issue_gather(c0 + 1, 1)

            finish_chunk(c0, 0)

            @pl.when(c0 + 2 < n_chunks)
            def _():
                drain_out(c0 + 2 - 2, 0)
                issue_gather(c0 + 2, 0)

            pl.when(c0 + 1 < n_chunks)(lambda: finish_chunk(c0 + 1, 1))
            return 0

        lax.fori_loop(0, (n_chunks + 1) // 2, pair_body, 0)
        drain_out(n_chunks - 2, (n_chunks - 2) % 2)
        drain_out(n_chunks - 1, (n_chunks - 1) % 2)

    return gather_kernel


def _tc_transpose(x, b, l):
    bb, lb = 512, 8

    def body(x_ref, o_ref):
        o_ref[...] = jnp.transpose(x_ref[:, :, 0:D], (1, 2, 0))

    return pl.pallas_call(
        body,
        grid=(l // lb, b // bb),
        in_specs=[pl.BlockSpec((bb, lb, 2 * D), lambda i, j: (j, i, 0))],
        out_specs=pl.BlockSpec((lb, D, bb), lambda i, j: (i, 0, j)),
        out_shape=jax.ShapeDtypeStruct((l, D, b), jnp.float32),
    )(x)


def kernel(x, table):
    b, l, _ = x.shape
    n = b * l
    idx = x[..., 0].astype(jnp.int32).reshape(n // IB, IB)
    dur = x[..., 1].reshape(n)
    tpad = jnp.pad(table, ((0, 0), (0, 1)))
    out = _make_gather(n)(tpad, idx, dur)
    out3 = _tc_transpose(out.reshape(b, l, 2 * D), b, l)
    return jnp.transpose(out3, (2, 0, 1))


# speedup vs baseline: 1.9416x; 1.9416x over previous
import functools

import jax
import jax.numpy as jnp
from jax import lax
from jax.experimental import pallas as pl
from jax.experimental.pallas import tpu as pltpu
from jax.experimental.pallas import tpu_sc as plsc

D = 64
BW = 128


def _make_kernel(b_total: int, l_total: int):
    info = plsc.get_sparse_core_info()
    nc, ns = info.num_cores, info.num_subcores
    nw = nc * ns
    assert b_total == nw * BW
    mesh = plsc.VectorSubcoreMesh(core_axis_name="c", subcore_axis_name="s")

    @functools.partial(
        pl.kernel,
        out_type=jax.ShapeDtypeStruct((l_total, 8, nw, 8 * BW), jnp.float32),
        mesh=mesh,
        compiler_params=pltpu.CompilerParams(
            use_tc_tiling_on_sc=False, needs_layout_passes=False),
        scratch_types=[
            pltpu.VMEM((l_total, BW), jnp.int32),
            pltpu.VMEM((l_total, BW), jnp.float32),
            pltpu.VMEM((BW, D), jnp.float32),
            pltpu.VMEM((BW, D), jnp.float32),
            pltpu.VMEM((8, 8 * BW), jnp.float32),
            pltpu.VMEM((8, 8 * BW), jnp.float32),
            pltpu.SemaphoreType.DMA,
            pltpu.SemaphoreType.DMA,
            pltpu.SemaphoreType.DMA,
            pltpu.SemaphoreType.DMA,
        ],
    )
    def sc_kernel(tpad_hbm, idxt_hbm, durt_hbm, out_hbm, idx_v, dur_v,
                  rows0, rows1, st0, st1, gsem0, gsem1, osem0, osem1):
        w = lax.axis_index("s") * nc + lax.axis_index("c")
        boff = pl.multiple_of(w * BW, BW)
        rows = (rows0, rows1)
        st = (st0, st1)
        gsem = (gsem0, gsem1)
        osem = (osem0, osem1)

        pltpu.sync_copy(idxt_hbm.at[:, pl.ds(boff, BW)], idx_v)
        pltpu.sync_copy(durt_hbm.at[:, pl.ds(boff, BW)], dur_v)

        def issue_gather(l, bi):
            pltpu.async_copy(tpad_hbm.at[idx_v.at[l]], rows[bi], gsem[bi])

        def drain_gather(bi):
            pltpu.make_async_copy(
                tpad_hbm.at[pl.ds(0, BW)], rows[bi], gsem[bi]).wait()

        def drain_out(bi):
            pltpu.make_async_copy(
                st[bi], out_hbm.at[0, :, 0], osem[bi]).wait()

        bvecs = [lax.iota(jnp.int32, 16) + bg * 16 for bg in range(8)]

        def transpose_rows(l, bi):
            group = 4
            for c0 in range(0, D - 1, group):
                cs = range(c0, min(c0 + group, D - 1))
                vals = [
                    plsc.load_gather(rows[bi],
                                     [bvecs[bg], jnp.full((16,), c,
                                                          jnp.int32)])
                    for c in cs for bg in range(8)
                ]
                i = 0
                for c in cs:
                    for bg in range(8):
                        st[bi][c // 8, pl.ds((c % 8) * BW + bg * 16, 16)] = (
                            vals[i])
                        i += 1
            for bg in range(8):
                st[bi][7, pl.ds(7 * BW + bg * 16, 16)] = (
                    dur_v[l, pl.ds(bg * 16, 16)])

        def step(l, bi):
            pl.when(l >= 2)(lambda: drain_out(bi))
            pl.when(l + 1 < l_total)(lambda: issue_gather(l + 1, 1 - bi))
            drain_gather(bi)
            plsc.subcore_barrier()
            transpose_rows(l, bi)
            pltpu.async_copy(st[bi], out_hbm.at[l, :, w], osem[bi])

        issue_gather(0, 0)

        def pair_body(p, _):
            step(p * 2, 0)
            step(p * 2 + 1, 1)
            return 0

        lax.fori_loop(0, l_total // 2, pair_body, 0)
        drain_out(0)
        drain_out(1)

    return sc_kernel


def kernel(x, table):
    b, l, _ = x.shape
    idxt = x[..., 0].astype(jnp.int32).T
    durt = x[..., 1].T
    tpad = jnp.pad(table, ((0, 0), (0, 1)))
    out4 = _make_kernel(b, l)(tpad, idxt, durt)
    nw = out4.shape[2]
    return (out4.reshape(l, 8, nw, 8, BW)
            .transpose(2, 4, 0, 1, 3)
            .reshape(b, l, D))


# speedup vs baseline: 5.8105x; 2.9926x over previous
import functools

import jax
import jax.numpy as jnp
from jax import lax
from jax.experimental import pallas as pl
from jax.experimental.pallas import tpu as pltpu
from jax.experimental.pallas import tpu_sc as plsc

D = 64
BW = 128


def _make_kernel(b_total: int, l_total: int):
    info = plsc.get_sparse_core_info()
    nc, ns = info.num_cores, info.num_subcores
    nw = nc * ns
    assert b_total == nw * BW
    mesh = plsc.VectorSubcoreMesh(core_axis_name="c", subcore_axis_name="s")

    @functools.partial(
        pl.kernel,
        out_type=jax.ShapeDtypeStruct((l_total, 8, nw, 8 * BW), jnp.float32),
        mesh=mesh,
        compiler_params=pltpu.CompilerParams(
            use_tc_tiling_on_sc=False, needs_layout_passes=False),
        scratch_types=[
            pltpu.VMEM((l_total, BW), jnp.int32),
            pltpu.VMEM((l_total, BW), jnp.float32),
            pltpu.VMEM((BW, D), jnp.float32),
            pltpu.VMEM((BW, D), jnp.float32),
            pltpu.VMEM((8, 8 * BW), jnp.float32),
            pltpu.VMEM((8, 8 * BW), jnp.float32),
            pltpu.SemaphoreType.DMA,
            pltpu.SemaphoreType.DMA,
            pltpu.SemaphoreType.DMA,
            pltpu.SemaphoreType.DMA,
        ],
    )
    def sc_kernel(tpad_hbm, idxt_hbm, durt_hbm, out_hbm, idx_v, dur_v,
                  rows0, rows1, st0, st1, gsem0, gsem1, osem0, osem1):
        w = lax.axis_index("s") * nc + lax.axis_index("c")
        boff = pl.multiple_of(w * BW, BW)
        rows = (rows0, rows1)
        st = (st0, st1)
        gsem = (gsem0, gsem1)
        osem = (osem0, osem1)

        pltpu.sync_copy(idxt_hbm.at[:, pl.ds(boff, BW)], idx_v)
        pltpu.sync_copy(durt_hbm.at[:, pl.ds(boff, BW)], dur_v)

        def issue_gather(l, bi):
            pltpu.async_copy(tpad_hbm.at[idx_v.at[l]], rows[bi], gsem[bi])

        def drain_gather(bi):
            pltpu.make_async_copy(
                tpad_hbm.at[pl.ds(0, BW)], rows[bi], gsem[bi]).wait()

        def drain_out(bi):
            pltpu.make_async_copy(
                st[bi], out_hbm.at[0, :, 0], osem[bi]).wait()

        jvec = lax.iota(jnp.int32, 16)
        q8vec = (jvec % 8) * BW
        rvecs = {c0: jvec // 8 + (c0 // 8) for c0 in range(0, D, 16)}
        cvecs = {c0: jvec + c0 for c0 in range(0, D, 16)}

        def transpose_rows(l, bi):
            def d_body(d, _):
                bdvec = (jvec + d) % 16
                for c0 in range(0, D, 16):
                    bidx = [bdvec + bg * 16 for bg in range(8)]
                    vals = [
                        plsc.load_gather(rows[bi], [bidx[bg], cvecs[c0]])
                        for bg in range(8)
                    ]
                    for bg in range(8):
                        plsc.store_scatter(
                            st[bi], [rvecs[c0], q8vec + bidx[bg]], vals[bg])
                return 0
            lax.fori_loop(0, 16, d_body, 0)
            for bg in range(8):
                st[bi][7, pl.ds(7 * BW + bg * 16, 16)] = (
                    dur_v[l, pl.ds(bg * 16, 16)])

        def step(l, bi):
            pl.when(l >= 2)(lambda: drain_out(bi))
            pl.when(l + 1 < l_total)(lambda: issue_gather(l + 1, 1 - bi))
            drain_gather(bi)
            plsc.subcore_barrier()
            transpose_rows(l, bi)
            pltpu.async_copy(st[bi], out_hbm.at[l, :, w], osem[bi])

        issue_gather(0, 0)

        def pair_body(p, _):
            step(p * 2, 0)
            step(p * 2 + 1, 1)
            return 0

        lax.fori_loop(0, l_total // 2, pair_body, 0)
        drain_out(0)
        drain_out(1)

    return sc_kernel


def kernel(x, table):
    b, l, _ = x.shape
    idxt = x[..., 0].astype(jnp.int32).T
    durt = x[..., 1].T
    tpad = jnp.pad(table, ((0, 0), (0, 1)))
    out4 = _make_kernel(b, l)(tpad, idxt, durt)
    nw = out4.shape[2]
    return (out4.reshape(l, 8, nw, 8, BW)
            .transpose(2, 4, 0, 1, 3)
            .reshape(b, l, D))


# speedup vs baseline: 6.2032x; 1.0676x over previous
import functools

import jax
import jax.numpy as jnp
from jax import lax
from jax.experimental import pallas as pl
from jax.experimental.pallas import tpu as pltpu
from jax.experimental.pallas import tpu_sc as plsc

D = 64
BW = 128


def _make_kernel(b_total: int, l_total: int):
    info = plsc.get_sparse_core_info()
    nc, ns = info.num_cores, info.num_subcores
    nw = nc * ns
    assert b_total == nw * BW
    mesh = plsc.VectorSubcoreMesh(core_axis_name="c", subcore_axis_name="s")

    @functools.partial(
        pl.kernel,
        out_type=jax.ShapeDtypeStruct((l_total, 8, nw, 8 * BW), jnp.float32),
        mesh=mesh,
        compiler_params=pltpu.CompilerParams(
            use_tc_tiling_on_sc=False, needs_layout_passes=False),
        scratch_types=[
            pltpu.VMEM((l_total, BW), jnp.int32),
            pltpu.VMEM((l_total, BW), jnp.float32),
            pltpu.VMEM((BW, D), jnp.float32),
            pltpu.VMEM((BW, D), jnp.float32),
            pltpu.VMEM((8, 8 * BW), jnp.float32),
            pltpu.VMEM((8, 8 * BW), jnp.float32),
            pltpu.SemaphoreType.DMA,
            pltpu.SemaphoreType.DMA,
            pltpu.SemaphoreType.DMA,
            pltpu.SemaphoreType.DMA,
        ],
    )
    def sc_kernel(tpad_hbm, idxt_hbm, durt_hbm, out_hbm, idx_v, dur_v,
                  rows0, rows1, st0, st1, gsem0, gsem1, osem0, osem1):
        w = lax.axis_index("s") * nc + lax.axis_index("c")
        boff = pl.multiple_of(w * BW, BW)
        rows = (rows0, rows1)
        st = (st0, st1)
        gsem = (gsem0, gsem1)
        osem = (osem0, osem1)

        pltpu.sync_copy(idxt_hbm.at[:, pl.ds(boff, BW)], idx_v)
        pltpu.sync_copy(durt_hbm.at[:, pl.ds(boff, BW)], dur_v)

        def issue_gather(l, bi):
            pltpu.async_copy(tpad_hbm.at[idx_v.at[l]], rows[bi], gsem[bi])

        def drain_gather(bi):
            pltpu.make_async_copy(
                tpad_hbm.at[pl.ds(0, BW)], rows[bi], gsem[bi]).wait()

        def drain_out(bi):
            pltpu.make_async_copy(
                st[bi], out_hbm.at[0, :, 0], osem[bi]).wait()

        jvec = lax.iota(jnp.int32, 16)
        q8vec = (jvec % 8) * BW
        rvecs = {c0: jvec // 8 + (c0 // 8) for c0 in range(0, D, 16)}
        cvecs = {c0: jvec + c0 for c0 in range(0, D, 16)}

        def transpose_rows(l, bi):
            def d_body(d, _):
                bdvec = (jvec + d) % 16
                for c0 in range(0, D, 16):
                    bidx = [bdvec + bg * 16 for bg in range(8)]
                    vals = [
                        plsc.load_gather(rows[bi], [bidx[bg], cvecs[c0]])
                        for bg in range(8)
                    ]
                    for bg in range(8):
                        plsc.store_scatter(
                            st[bi], [rvecs[c0], q8vec + bidx[bg]], vals[bg])
                return 0
            lax.fori_loop(0, 16, d_body, 0)
            for bg in range(8):
                st[bi][7, pl.ds(7 * BW + bg * 16, 16)] = (
                    dur_v[l, pl.ds(bg * 16, 16)])

        def step(l, bi):
            pl.when(l >= 2)(lambda: drain_out(bi))
            pl.when(l + 1 < l_total)(lambda: issue_gather(l + 1, 1 - bi))
            drain_gather(bi)
            transpose_rows(l, bi)
            pltpu.async_copy(st[bi], out_hbm.at[l, :, w], osem[bi])

        issue_gather(0, 0)

        def pair_body(p, _):
            step(p * 2, 0)
            step(p * 2 + 1, 1)
            return 0

        lax.fori_loop(0, l_total // 2, pair_body, 0)
        drain_out(0)
        drain_out(1)

    return sc_kernel


def kernel(x, table):
    b, l, _ = x.shape
    idxt = x[..., 0].astype(jnp.int32).T
    durt = x[..., 1].T
    tpad = jnp.pad(table, ((0, 0), (0, 1)))
    out4 = _make_kernel(b, l)(tpad, idxt, durt)
    nw = out4.shape[2]
    return (out4.reshape(l, 8, nw, 8, BW)
            .transpose(2, 4, 0, 1, 3)
            .reshape(b, l, D))
